# baseline (speedup 1.0000x reference)
import functools

import jax
import jax.numpy as jnp
from jax import lax
from jax.experimental import pallas as pl
from jax.experimental.pallas import tpu as pltpu
from jax.experimental.pallas import tpu_sc as plsc

_N = 25000
_NTILES = 16
_TROWS = 1664
_NACC = _NTILES * _TROWS
_E = 400000
_CW = 128
_NW = 32
_NBUF = 4
_CHUNKS = 100
_EPAD = _NW * _CHUNKS * _CW
_R = 1000
_GRID = _N // _R

_SC_PARAMS = pltpu.CompilerParams(use_tc_tiling_on_sc=False)
_MESH = dict(core_axis_name="c", subcore_axis_name="s")


def _make_agg1(D, dtype=jnp.float32):
    @functools.partial(
        pl.kernel,
        out_type=jax.ShapeDtypeStruct((2, _NACC, D), dtype),
        mesh=plsc.VectorSubcoreMesh(**_MESH),
        compiler_params=_SC_PARAMS,
        scratch_types=[
            pltpu.VMEM((_CHUNKS, _CW), jnp.int32),
            pltpu.VMEM((_CHUNKS, _CW), jnp.int32),
            [pltpu.VMEM((_CW, D), dtype)] * _NBUF,
            [pltpu.SemaphoreType.DMA] * _NBUF,
            [pltpu.SemaphoreType.DMA] * _NBUF,
            pltpu.VMEM_SHARED((_NACC, D), dtype),
        ],
    )
    def agg(xs_hbm, src_hbm, dst_hbm, zero_hbm, out_hbm, srcb, dstb, gbufs,
            gsems, ssems, acc):
        cid = lax.axis_index("c")
        sid = lax.axis_index("s")
        wid = cid * _NTILES + sid
        lo = sid * _TROWS
        pltpu.sync_copy(src_hbm.at[wid], srcb)
        pltpu.sync_copy(dst_hbm.at[wid], dstb)
        pltpu.sync_copy(zero_hbm, acc.at[pl.ds(lo, _TROWS)])
        plsc.subcore_barrier()

        def gath(c, b):
            pltpu.async_copy(xs_hbm.at[srcb.at[c]], gbufs[b], gsems[b])

        def gath_wait(c, b):
            pltpu.make_async_copy(xs_hbm.at[srcb.at[c]], gbufs[b],
                                  gsems[b]).wait()

        def scat(c, b):
            pltpu.async_copy(gbufs[b], acc.at[dstb.at[c]], ssems[b],
                             add=True)

        def scat_wait(c, b):
            pltpu.make_async_copy(gbufs[b], acc.at[dstb.at[c]],
                                  ssems[b]).wait()

        for b in range(_NBUF):
            gath(b, b)

        def body(i, carry):
            c0 = i * _NBUF
            for b in range(_NBUF):
                gath_wait(c0 + b, b)
                scat(c0 + b, b)
            for b in range(_NBUF):
                scat_wait(c0 + b, b)
                gath(c0 + _NBUF + b, b)
            return carry

        lax.fori_loop(0, _CHUNKS // _NBUF - 1, body, 0)
        c0 = _CHUNKS - _NBUF
        for b in range(_NBUF):
            gath_wait(c0 + b, b)
            scat(c0 + b, b)
        for b in range(_NBUF):
            scat_wait(c0 + b, b)
        plsc.subcore_barrier()
        pltpu.sync_copy(acc.at[pl.ds(lo, _TROWS)],
                        out_hbm.at[cid, pl.ds(lo, _TROWS)])

    return agg


def _make_deg(D=8):
    @functools.partial(
        pl.kernel,
        out_type=jax.ShapeDtypeStruct((2, _NACC, D), jnp.float32),
        mesh=plsc.VectorSubcoreMesh(**_MESH),
        compiler_params=_SC_PARAMS,
        scratch_types=[
            pltpu.VMEM((_CHUNKS, _CW), jnp.int32),
            pltpu.VMEM((_CW, D), jnp.float32),
            pltpu.VMEM_SHARED((_NACC, D), jnp.float32),
        ],
    )
    def deg(ones_hbm, dst_hbm, zero_hbm, out_hbm, dstb, obuf, acc):
        cid = lax.axis_index("c")
        sid = lax.axis_index("s")
        wid = cid * _NTILES + sid
        lo = sid * _TROWS
        pltpu.sync_copy(dst_hbm.at[wid], dstb)
        pltpu.sync_copy(ones_hbm, obuf)
        pltpu.sync_copy(zero_hbm, acc.at[pl.ds(lo, _TROWS)])
        plsc.subcore_barrier()

        def body(c, carry):
            pltpu.sync_copy(obuf, acc.at[dstb.at[c]], add=True)
            return carry

        lax.fori_loop(0, _CHUNKS, body, 0)
        plsc.subcore_barrier()
        pltpu.sync_copy(acc.at[pl.ds(lo, _TROWS)],
                        out_hbm.at[cid, pl.ds(lo, _TROWS)])

    return deg


def _make_block(Din, Dout, F=0):
    def body(*refs):
        if F:
            (m_ref, w_ref, b_ref, k0, k1, k2, kb_ref, wfc_ref, bfc_ref,
             out_ref, acc) = refs
        else:
            m_ref, w_ref, b_ref, k0, k1, k2, kb_ref, out_ref = refs
        g = jnp.dot(m_ref[...], w_ref[...],
                    preferred_element_type=jnp.float32)
        g = g + b_ref[...]
        t = lax.broadcasted_iota(jnp.int32, (_R, 1), 0) % 10
        bf = jnp.bfloat16
        gb = g.astype(bf)
        zrow = jnp.zeros((1, Dout), bf)
        gm1 = jnp.where(t == 0, jnp.zeros((), bf),
                        jnp.concatenate([zrow, gb[:-1]], axis=0))
        gp1 = jnp.where(t == 9, jnp.zeros((), bf),
                        jnp.concatenate([gb[1:], zrow], axis=0))
        y = (jnp.dot(gb, k1[...], preferred_element_type=jnp.float32)
             + jnp.dot(gm1, k0[...], preferred_element_type=jnp.float32)
             + jnp.dot(gp1, k2[...], preferred_element_type=jnp.float32)
             + kb_ref[...])
        y = jnp.maximum(y, 0.0)
        if F:
            i = pl.program_id(0)

            @pl.when(i == 0)
            def _():
                acc[...] = jnp.zeros_like(acc)

            acc[...] += jnp.sum(y.reshape(_R // 8, 8, Dout), axis=0)

            @pl.when(i == _GRID - 1)
            def _():
                cm = jnp.sum(acc[...], axis=0, keepdims=True) * (1.0 / _N)
                out_ref[...] = (jnp.dot(cm, wfc_ref[...],
                                        preferred_element_type=jnp.float32)
                                + bfc_ref[...])
        else:
            out_ref[...] = y.astype(bf)

    full = lambda i: (0, 0)
    in_specs = [
        pl.BlockSpec((_R, Din), lambda i: (i, 0)),
        pl.BlockSpec((Din, Dout), full),
        pl.BlockSpec((1, Dout), full),
        pl.BlockSpec((Dout, Dout), full),
        pl.BlockSpec((Dout, Dout), full),
        pl.BlockSpec((Dout, Dout), full),
        pl.BlockSpec((1, Dout), full),
    ]
    scratch = []
    if F:
        in_specs += [pl.BlockSpec((Dout, F), full),
                     pl.BlockSpec((1, F), full)]
        out_specs = [pl.BlockSpec((1, F), full)]
        out_shape = [jax.ShapeDtypeStruct((1, F), jnp.float32)]
        scratch = [pltpu.VMEM((8, Dout), jnp.float32)]
    else:
        out_specs = [pl.BlockSpec((_R, Dout), lambda i: (i, 0))]
        out_shape = [jax.ShapeDtypeStruct((_N, Dout), jnp.bfloat16)]

    def run(*args):
        return pl.pallas_call(
            body,
            grid=(_GRID,),
            in_specs=in_specs,
            out_specs=out_specs,
            out_shape=out_shape,
            scratch_shapes=scratch,
        )(*args)

    return run


_deg_k = _make_deg()
_agg16 = _make_agg1(16)
_agg64b = _make_agg1(64, jnp.bfloat16)
_blk1 = _make_block(16, 64)
_blk2 = _make_block(64, 128)


def kernel(x, edge_index, W1, b1, K1, kb1, W2, b2, K2, kb2, W3, b3, K3, kb3,
           Wfc, bfc):
    f32, bf = jnp.float32, jnp.bfloat16
    xf = x.reshape(_N, 16)
    src = edge_index[0].astype(jnp.int32)
    dst = edge_index[1].astype(jnp.int32)
    pad = _EPAD - _E
    padi = jnp.arange(pad, dtype=jnp.int32)
    src_p = jnp.concatenate([src, padi % _N]).reshape(_NW, _CHUNKS, _CW)
    dst_p = jnp.concatenate(
        [dst, _N + 600 + (padi % 1024)]).reshape(_NW, _CHUNKS, _CW)

    ones8 = jnp.ones((_CW, 8), f32)
    z8 = jnp.zeros((_TROWS, 8), f32)
    z16 = jnp.zeros((_TROWS, 16), f32)
    z64b = jnp.zeros((_TROWS, 64), bf)

    deg8 = _deg_k(ones8, dst_p, z8)
    deg = deg8[0, :_N, 0:1] + deg8[1, :_N, 0:1] + 1.0
    dis = lax.rsqrt(deg)
    xs1 = dis * xf

    kt = lambda K, h: K[:, :, h, 0].T.astype(bf)
    rs = lambda v: v.reshape(1, -1)

    agg1 = _agg16(xs1, src_p, dst_p, z16)
    m1 = (dis * (agg1[0, :_N] + agg1[1, :_N] + xs1)).astype(bf)
    y2 = _blk1(m1, W1.astype(bf), rs(b1), kt(K1, 0), kt(K1, 1), kt(K1, 2),
               rs(kb1))[0]

    ys2 = (dis * y2.astype(f32)).astype(bf)
    ag2 = _agg64b(ys2, src_p, dst_p, z64b)
    m2 = (dis * (ag2[0, :_N].astype(f32) + ag2[1, :_N].astype(f32)
                 + ys2.astype(f32))).astype(bf)
    y3 = _blk2(m2, W2.astype(bf), rs(b2), kt(K2, 0), kt(K2, 1), kt(K2, 2),
               rs(kb2))[0]

    ys3 = dis * y3.astype(f32)
    ys3a = ys3[:, :64].astype(bf)
    ys3b = ys3[:, 64:].astype(bf)
    ag3a = _agg64b(ys3a, src_p, dst_p, z64b)
    ag3b = _agg64b(ys3b, src_p, dst_p, z64b)
    a3 = jnp.concatenate(
        [ag3a[0, :_N].astype(f32) + ag3a[1, :_N].astype(f32),
         ag3b[0, :_N].astype(f32) + ag3b[1, :_N].astype(f32)], axis=1)
    m3 = (dis * (a3 + ys3)).astype(bf)

    blk3 = _make_block(128, 256, F=Wfc.shape[1])
    out = blk3(m3, W3.astype(bf), rs(b3), kt(K3, 0), kt(K3, 1), kt(K3, 2),
               rs(kb3), Wfc, rs(bfc))[0]
    return out


# speedup vs baseline: 1.0966x; 1.0966x over previous
import functools

import jax
import jax.numpy as jnp
from jax import lax
from jax.experimental import pallas as pl
from jax.experimental.pallas import tpu as pltpu
from jax.experimental.pallas import tpu_sc as plsc

_N = 25000
_NTILES = 16
_TROWS = 1664
_NACC = _NTILES * _TROWS
_E = 400000
_CW = 128
_NW = 32
_NBUF = 4
_CHUNKS = 100
_EPAD = _NW * _CHUNKS * _CW
_R = 1000
_GRID = _N // _R

_SC_PARAMS = pltpu.CompilerParams(use_tc_tiling_on_sc=False)
_MESH = dict(core_axis_name="c", subcore_axis_name="s")


def _make_agg1(D, dtype=jnp.float32):
    @functools.partial(
        pl.kernel,
        out_type=jax.ShapeDtypeStruct((2, _NACC, D), dtype),
        mesh=plsc.VectorSubcoreMesh(**_MESH),
        compiler_params=_SC_PARAMS,
        scratch_types=[
            pltpu.VMEM((_CHUNKS, _CW), jnp.int32),
            pltpu.VMEM((_CHUNKS, _CW), jnp.int32),
            [pltpu.VMEM((_CW, D), dtype)] * _NBUF,
            [pltpu.SemaphoreType.DMA] * _NBUF,
            [pltpu.SemaphoreType.DMA] * _NBUF,
            pltpu.VMEM_SHARED((_NACC, D), dtype),
        ],
    )
    def agg(xs_hbm, src_hbm, dst_hbm, zero_hbm, out_hbm, srcb, dstb, gbufs,
            gsems, ssems, acc):
        cid = lax.axis_index("c")
        sid = lax.axis_index("s")
        wid = cid * _NTILES + sid
        lo = sid * _TROWS
        pltpu.sync_copy(src_hbm.at[wid], srcb)
        pltpu.sync_copy(dst_hbm.at[wid], dstb)
        pltpu.sync_copy(zero_hbm, acc.at[pl.ds(lo, _TROWS)])
        plsc.subcore_barrier()

        def gath(c, b):
            pltpu.async_copy(xs_hbm.at[srcb.at[c]], gbufs[b], gsems[b])

        def gath_wait(c, b):
            pltpu.make_async_copy(xs_hbm.at[srcb.at[c]], gbufs[b],
                                  gsems[b]).wait()

        def scat(c, b):
            pltpu.async_copy(gbufs[b], acc.at[dstb.at[c]], ssems[b],
                             add=True)

        def scat_wait(c, b):
            pltpu.make_async_copy(gbufs[b], acc.at[dstb.at[c]],
                                  ssems[b]).wait()

        for b in range(_NBUF):
            gath(b, b)

        def body(i, carry):
            c0 = i * _NBUF
            for b in range(_NBUF):
                gath_wait(c0 + b, b)
                scat(c0 + b, b)
            for b in range(_NBUF):
                scat_wait(c0 + b, b)
                gath(c0 + _NBUF + b, b)
            return carry

        lax.fori_loop(0, _CHUNKS // _NBUF - 1, body, 0)
        c0 = _CHUNKS - _NBUF
        for b in range(_NBUF):
            gath_wait(c0 + b, b)
            scat(c0 + b, b)
        for b in range(_NBUF):
            scat_wait(c0 + b, b)
        plsc.subcore_barrier()
        pltpu.sync_copy(acc.at[pl.ds(lo, _TROWS)],
                        out_hbm.at[cid, pl.ds(lo, _TROWS)])

    return agg


def _make_deg(D=8):
    @functools.partial(
        pl.kernel,
        out_type=jax.ShapeDtypeStruct((2, _NACC, D), jnp.float32),
        mesh=plsc.VectorSubcoreMesh(**_MESH),
        compiler_params=_SC_PARAMS,
        scratch_types=[
            pltpu.VMEM((_CHUNKS, _CW), jnp.int32),
            pltpu.VMEM((_CW, D), jnp.float32),
            pltpu.VMEM_SHARED((_NACC, D), jnp.float32),
        ],
    )
    def deg(ones_hbm, dst_hbm, zero_hbm, out_hbm, dstb, obuf, acc):
        cid = lax.axis_index("c")
        sid = lax.axis_index("s")
        wid = cid * _NTILES + sid
        lo = sid * _TROWS
        pltpu.sync_copy(dst_hbm.at[wid], dstb)
        pltpu.sync_copy(ones_hbm, obuf)
        pltpu.sync_copy(zero_hbm, acc.at[pl.ds(lo, _TROWS)])
        plsc.subcore_barrier()

        def body(c, carry):
            pltpu.sync_copy(obuf, acc.at[dstb.at[c]], add=True)
            return carry

        lax.fori_loop(0, _CHUNKS, body, 0)
        plsc.subcore_barrier()
        pltpu.sync_copy(acc.at[pl.ds(lo, _TROWS)],
                        out_hbm.at[cid, pl.ds(lo, _TROWS)])

    return deg


def _dis_body(deg_ref, x_ref, dis_ref, xs_ref):
    d = deg_ref[0, :, 0:1] + deg_ref[1, :, 0:1] + 1.0
    dis = lax.rsqrt(d)
    dis_ref[...] = dis
    xs_ref[...] = dis * x_ref[...]


def _dis_kernel(deg_out, xf):
    return pl.pallas_call(
        _dis_body,
        grid=(_GRID,),
        in_specs=[
            pl.BlockSpec((2, _R, 8), lambda i: (0, i, 0)),
            pl.BlockSpec((_R, 16), lambda i: (i, 0)),
        ],
        out_specs=[
            pl.BlockSpec((_R, 1), lambda i: (i, 0)),
            pl.BlockSpec((_R, 16), lambda i: (i, 0)),
        ],
        out_shape=[
            jax.ShapeDtypeStruct((_N, 1), jnp.float32),
            jax.ShapeDtypeStruct((_N, 16), jnp.float32),
        ],
    )(deg_out, xf)


def _make_block(Din, Dout, n_in, n_out, Dc=None, F=0):
    Dc = Dc or Dout
    bf = jnp.bfloat16
    f32 = jnp.float32

    def body(*refs):
        dis_ref = refs[0]
        xs_refs = refs[1:1 + n_in]
        ag_refs = refs[1 + n_in:1 + 2 * n_in]
        w_ref, b_ref, k0, k1, k2, kb_ref = refs[1 + 2 * n_in:7 + 2 * n_in]
        rest = refs[7 + 2 * n_in:]
        if n_in == 1:
            xs = xs_refs[0][...].astype(f32)
            a = ag_refs[0][0].astype(f32) + ag_refs[0][1].astype(f32)
        else:
            xs = jnp.concatenate(
                [r[...].astype(f32) for r in xs_refs], axis=-1)
            a = jnp.concatenate(
                [r[0].astype(f32) + r[1].astype(f32) for r in ag_refs],
                axis=-1)
        dis = dis_ref[...]
        m = dis * (a + xs)
        g = jnp.dot(m.astype(bf), w_ref[...],
                    preferred_element_type=f32)
        g = g + b_ref[...]
        t = lax.broadcasted_iota(jnp.int32, (_R, 1), 0) % 10
        gb = g.astype(bf)
        zrow = jnp.zeros((1, Dout), bf)
        gm1 = jnp.where(t == 0, jnp.zeros((), bf),
                        jnp.concatenate([zrow, gb[:-1]], axis=0))
        gp1 = jnp.where(t == 9, jnp.zeros((), bf),
                        jnp.concatenate([gb[1:], zrow], axis=0))
        y = (jnp.dot(gb, k1[...], preferred_element_type=f32)
             + jnp.dot(gm1, k0[...], preferred_element_type=f32)
             + jnp.dot(gp1, k2[...], preferred_element_type=f32)
             + kb_ref[...])
        y = jnp.maximum(y, 0.0)
        if F:
            wfc_ref, bfc_ref, out_ref, acc = rest
            i = pl.program_id(0)

            @pl.when(i == 0)
            def _():
                acc[...] = jnp.zeros_like(acc)

            acc[...] += jnp.sum(y.reshape(_R // 8, 8, Dc), axis=0)

            @pl.when(i == _GRID - 1)
            def _():
                cm = jnp.sum(acc[...], axis=0, keepdims=True) * (1.0 / _N)
                out_ref[...] = (jnp.dot(cm, wfc_ref[...],
                                        preferred_element_type=f32)
                                + bfc_ref[...])
        else:
            ys = (dis * y).astype(bf)
            q = Dc // n_out
            for j in range(n_out):
                rest[j][...] = ys[:, j * q:(j + 1) * q]

    P = Din // n_in
    full = lambda i: (0, 0)
    in_specs = [pl.BlockSpec((_R, 1), lambda i: (i, 0))]
    in_specs += [pl.BlockSpec((_R, P), lambda i: (i, 0))] * n_in
    in_specs += [pl.BlockSpec((2, _R, P), lambda i: (0, i, 0))] * n_in
    in_specs += [
        pl.BlockSpec((Din, Dout), full),
        pl.BlockSpec((1, Dout), full),
        pl.BlockSpec((Dout, Dc), full),
        pl.BlockSpec((Dout, Dc), full),
        pl.BlockSpec((Dout, Dc), full),
        pl.BlockSpec((1, Dc), full),
    ]
    scratch = []
    if F:
        in_specs += [pl.BlockSpec((Dc, F), full), pl.BlockSpec((1, F), full)]
        out_specs = [pl.BlockSpec((1, F), full)]
        out_shape = [jax.ShapeDtypeStruct((1, F), f32)]
        scratch = [pltpu.VMEM((8, Dc), f32)]
    else:
        q = Dc // n_out
        out_specs = [pl.BlockSpec((_R, q), lambda i: (i, 0))] * n_out
        out_shape = [jax.ShapeDtypeStruct((_N, q), bf)] * n_out

    def run(*args):
        return pl.pallas_call(
            body,
            grid=(_GRID,),
            in_specs=in_specs,
            out_specs=out_specs,
            out_shape=out_shape,
            scratch_shapes=scratch,
        )(*args)

    return run


_deg_k = _make_deg()
_agg16 = _make_agg1(16)
_agg64b = _make_agg1(64, jnp.bfloat16)
_blk1 = _make_block(16, 64, 1, 1)
_blk2h = _make_block(64, 128, 1, 1, Dc=64)


def kernel(x, edge_index, W1, b1, K1, kb1, W2, b2, K2, kb2, W3, b3, K3, kb3,
           Wfc, bfc):
    f32, bf = jnp.float32, jnp.bfloat16
    xf = x.reshape(_N, 16)
    src = edge_index[0].astype(jnp.int32)
    dst = edge_index[1].astype(jnp.int32)
    pad = _EPAD - _E
    padi = jnp.arange(pad, dtype=jnp.int32)
    src_p = jnp.concatenate([src, padi % _N]).reshape(_NW, _CHUNKS, _CW)
    dst_p = jnp.concatenate(
        [dst, _N + 600 + (padi % 1024)]).reshape(_NW, _CHUNKS, _CW)

    ones8 = jnp.ones((_CW, 8), f32)
    z8 = jnp.zeros((_TROWS, 8), f32)
    z16 = jnp.zeros((_TROWS, 16), f32)
    z64b = jnp.zeros((_TROWS, 64), bf)

    deg_out = _deg_k(ones8, dst_p, z8)
    dis, xs1 = _dis_kernel(deg_out, xf)

    kt = lambda K, h: K[:, :, h, 0].T.astype(bf)
    ktc = lambda K, h, j: K[:, :, h, 0].T[:, 64 * j:64 * (j + 1)].astype(bf)
    rs = lambda v: v.reshape(1, -1)

    agg1 = _agg16(xs1, src_p, dst_p, z16)
    ys2 = _blk1(dis, xs1, agg1, W1.astype(bf), rs(b1), kt(K1, 0), kt(K1, 1),
                kt(K1, 2), rs(kb1))[0]
    ag2 = _agg64b(ys2, src_p, dst_p, z64b)
    W2b = W2.astype(bf)
    ys3a = _blk2h(dis, ys2, ag2, W2b, rs(b2), ktc(K2, 0, 0), ktc(K2, 1, 0),
                  ktc(K2, 2, 0), rs(kb2)[:, :64])[0]
    ag3a = _agg64b(ys3a, src_p, dst_p, z64b)
    ys3b = _blk2h(dis, ys2, ag2, W2b, rs(b2), ktc(K2, 0, 1), ktc(K2, 1, 1),
                  ktc(K2, 2, 1), rs(kb2)[:, 64:])[0]
    ag3b = _agg64b(ys3b, src_p, dst_p, z64b)

    blk3 = _make_block(128, 256, 2, 0, F=Wfc.shape[1])
    out = blk3(dis, ys3a, ys3b, ag3a, ag3b, W3.astype(bf), rs(b3),
               kt(K3, 0), kt(K3, 1), kt(K3, 2), rs(kb3), Wfc, rs(bfc))[0]
    return out


# speedup vs baseline: 1.1411x; 1.0406x over previous
import functools

import jax
import jax.numpy as jnp
from jax import lax
from jax.experimental import pallas as pl
from jax.experimental.pallas import tpu as pltpu
from jax.experimental.pallas import tpu_sc as plsc

_N = 25000
_NTILES = 16
_TROWS = 1664
_NACC = _NTILES * _TROWS
_E = 400000
_CW = 128
_NW = 32
_NBUF = 10
_CHUNKS = 100
_EPAD = _NW * _CHUNKS * _CW
_R = 1000
_GRID = _N // _R

_SC_PARAMS = pltpu.CompilerParams(use_tc_tiling_on_sc=False)
_MESH = dict(core_axis_name="c", subcore_axis_name="s")


def _make_agg1(D, dtype=jnp.float32):
    @functools.partial(
        pl.kernel,
        out_type=jax.ShapeDtypeStruct((2, _NACC, D), dtype),
        mesh=plsc.VectorSubcoreMesh(**_MESH),
        compiler_params=_SC_PARAMS,
        scratch_types=[
            pltpu.VMEM((_CHUNKS, _CW), jnp.int32),
            pltpu.VMEM((_CHUNKS, _CW), jnp.int32),
            [pltpu.VMEM((_CW, D), dtype)] * _NBUF,
            [pltpu.SemaphoreType.DMA] * _NBUF,
            [pltpu.SemaphoreType.DMA] * _NBUF,
            pltpu.VMEM_SHARED((_NACC, D), dtype),
        ],
    )
    def agg(xs_hbm, src_hbm, dst_hbm, zero_hbm, out_hbm, srcb, dstb, gbufs,
            gsems, ssems, acc):
        cid = lax.axis_index("c")
        sid = lax.axis_index("s")
        wid = cid * _NTILES + sid
        lo = sid * _TROWS
        pltpu.sync_copy(src_hbm.at[wid], srcb)
        pltpu.sync_copy(dst_hbm.at[wid], dstb)
        pltpu.sync_copy(zero_hbm, acc.at[pl.ds(lo, _TROWS)])
        plsc.subcore_barrier()

        def gath(c, b):
            pltpu.async_copy(xs_hbm.at[srcb.at[c]], gbufs[b], gsems[b])

        def gath_wait(c, b):
            pltpu.make_async_copy(xs_hbm.at[srcb.at[c]], gbufs[b],
                                  gsems[b]).wait()

        def scat(c, b):
            pltpu.async_copy(gbufs[b], acc.at[dstb.at[c]], ssems[b],
                             add=True)

        def scat_wait(c, b):
            pltpu.make_async_copy(gbufs[b], acc.at[dstb.at[c]],
                                  ssems[b]).wait()

        for b in range(_NBUF):
            gath(b, b)

        def body(i, carry):
            c0 = i * _NBUF
            for b in range(_NBUF):
                gath_wait(c0 + b, b)
                scat(c0 + b, b)
            for b in range(_NBUF):
                scat_wait(c0 + b, b)
                gath(c0 + _NBUF + b, b)
            return carry

        lax.fori_loop(0, _CHUNKS // _NBUF - 1, body, 0)
        c0 = _CHUNKS - _NBUF
        for b in range(_NBUF):
            gath_wait(c0 + b, b)
            scat(c0 + b, b)
        for b in range(_NBUF):
            scat_wait(c0 + b, b)
        plsc.subcore_barrier()
        pltpu.sync_copy(acc.at[pl.ds(lo, _TROWS)],
                        out_hbm.at[cid, pl.ds(lo, _TROWS)])

    return agg


def _make_deg(D=8):
    @functools.partial(
        pl.kernel,
        out_type=jax.ShapeDtypeStruct((2, _NACC, D), jnp.float32),
        mesh=plsc.VectorSubcoreMesh(**_MESH),
        compiler_params=_SC_PARAMS,
        scratch_types=[
            pltpu.VMEM((_CHUNKS, _CW), jnp.int32),
            pltpu.VMEM((_CW, D), jnp.float32),
            pltpu.VMEM_SHARED((_NACC, D), jnp.float32),
        ],
    )
    def deg(ones_hbm, dst_hbm, zero_hbm, out_hbm, dstb, obuf, acc):
        cid = lax.axis_index("c")
        sid = lax.axis_index("s")
        wid = cid * _NTILES + sid
        lo = sid * _TROWS
        pltpu.sync_copy(dst_hbm.at[wid], dstb)
        pltpu.sync_copy(ones_hbm, obuf)
        pltpu.sync_copy(zero_hbm, acc.at[pl.ds(lo, _TROWS)])
        plsc.subcore_barrier()

        def body(c, carry):
            pltpu.sync_copy(obuf, acc.at[dstb.at[c]], add=True)
            return carry

        lax.fori_loop(0, _CHUNKS, body, 0)
        plsc.subcore_barrier()
        pltpu.sync_copy(acc.at[pl.ds(lo, _TROWS)],
                        out_hbm.at[cid, pl.ds(lo, _TROWS)])

    return deg


def _dis_body(deg_ref, x_ref, dis_ref, xs_ref):
    d = deg_ref[0, :, 0:1] + deg_ref[1, :, 0:1] + 1.0
    dis = lax.rsqrt(d)
    dis_ref[...] = dis
    xs_ref[...] = dis * x_ref[...]


def _dis_kernel(deg_out, xf):
    return pl.pallas_call(
        _dis_body,
        grid=(_GRID,),
        in_specs=[
            pl.BlockSpec((2, _R, 8), lambda i: (0, i, 0)),
            pl.BlockSpec((_R, 16), lambda i: (i, 0)),
        ],
        out_specs=[
            pl.BlockSpec((_R, 1), lambda i: (i, 0)),
            pl.BlockSpec((_R, 16), lambda i: (i, 0)),
        ],
        out_shape=[
            jax.ShapeDtypeStruct((_N, 1), jnp.float32),
            jax.ShapeDtypeStruct((_N, 16), jnp.float32),
        ],
    )(deg_out, xf)


def _make_block(Din, Dout, n_in, n_out, Dc=None, F=0):
    Dc = Dc or Dout
    bf = jnp.bfloat16
    f32 = jnp.float32

    def body(*refs):
        dis_ref = refs[0]
        xs_refs = refs[1:1 + n_in]
        ag_refs = refs[1 + n_in:1 + 2 * n_in]
        w_ref, b_ref, k0, k1, k2, kb_ref = refs[1 + 2 * n_in:7 + 2 * n_in]
        rest = refs[7 + 2 * n_in:]
        if n_in == 1:
            xs = xs_refs[0][...].astype(f32)
            a = ag_refs[0][0].astype(f32) + ag_refs[0][1].astype(f32)
        else:
            xs = jnp.concatenate(
                [r[...].astype(f32) for r in xs_refs], axis=-1)
            a = jnp.concatenate(
                [r[0].astype(f32) + r[1].astype(f32) for r in ag_refs],
                axis=-1)
        dis = dis_ref[...]
        m = dis * (a + xs)
        g = jnp.dot(m.astype(bf), w_ref[...],
                    preferred_element_type=f32)
        g = g + b_ref[...]
        t = lax.broadcasted_iota(jnp.int32, (_R, 1), 0) % 10
        gb = g.astype(bf)
        zrow = jnp.zeros((1, Dout), bf)
        gm1 = jnp.where(t == 0, jnp.zeros((), bf),
                        jnp.concatenate([zrow, gb[:-1]], axis=0))
        gp1 = jnp.where(t == 9, jnp.zeros((), bf),
                        jnp.concatenate([gb[1:], zrow], axis=0))
        y = (jnp.dot(gb, k1[...], preferred_element_type=f32)
             + jnp.dot(gm1, k0[...], preferred_element_type=f32)
             + jnp.dot(gp1, k2[...], preferred_element_type=f32)
             + kb_ref[...])
        y = jnp.maximum(y, 0.0)
        if F:
            wfc_ref, bfc_ref, out_ref, acc = rest
            i = pl.program_id(0)

            @pl.when(i == 0)
            def _():
                acc[...] = jnp.zeros_like(acc)

            acc[...] += jnp.sum(y.reshape(_R // 8, 8, Dc), axis=0)

            @pl.when(i == _GRID - 1)
            def _():
                cm = jnp.sum(acc[...], axis=0, keepdims=True) * (1.0 / _N)
                out_ref[...] = (jnp.dot(cm, wfc_ref[...],
                                        preferred_element_type=f32)
                                + bfc_ref[...])
        else:
            ys = (dis * y).astype(bf)
            q = Dc // n_out
            for j in range(n_out):
                rest[j][...] = ys[:, j * q:(j + 1) * q]

    P = Din // n_in
    full = lambda i: (0, 0)
    in_specs = [pl.BlockSpec((_R, 1), lambda i: (i, 0))]
    in_specs += [pl.BlockSpec((_R, P), lambda i: (i, 0))] * n_in
    in_specs += [pl.BlockSpec((2, _R, P), lambda i: (0, i, 0))] * n_in
    in_specs += [
        pl.BlockSpec((Din, Dout), full),
        pl.BlockSpec((1, Dout), full),
        pl.BlockSpec((Dout, Dc), full),
        pl.BlockSpec((Dout, Dc), full),
        pl.BlockSpec((Dout, Dc), full),
        pl.BlockSpec((1, Dc), full),
    ]
    scratch = []
    if F:
        in_specs += [pl.BlockSpec((Dc, F), full), pl.BlockSpec((1, F), full)]
        out_specs = [pl.BlockSpec((1, F), full)]
        out_shape = [jax.ShapeDtypeStruct((1, F), f32)]
        scratch = [pltpu.VMEM((8, Dc), f32)]
    else:
        q = Dc // n_out
        out_specs = [pl.BlockSpec((_R, q), lambda i: (i, 0))] * n_out
        out_shape = [jax.ShapeDtypeStruct((_N, q), bf)] * n_out

    def run(*args):
        return pl.pallas_call(
            body,
            grid=(_GRID,),
            in_specs=in_specs,
            out_specs=out_specs,
            out_shape=out_shape,
            scratch_shapes=scratch,
        )(*args)

    return run


_deg_k = _make_deg()
_agg16 = _make_agg1(16)
_agg64b = _make_agg1(64, jnp.bfloat16)
_blk1 = _make_block(16, 64, 1, 1)
_blk2 = _make_block(64, 128, 1, 2)


def kernel(x, edge_index, W1, b1, K1, kb1, W2, b2, K2, kb2, W3, b3, K3, kb3,
           Wfc, bfc):
    f32, bf = jnp.float32, jnp.bfloat16
    xf = x.reshape(_N, 16)
    src = edge_index[0].astype(jnp.int32)
    dst = edge_index[1].astype(jnp.int32)
    pad = _EPAD - _E
    padi = jnp.arange(pad, dtype=jnp.int32)
    src_p = jnp.concatenate([src, padi % _N]).reshape(_NW, _CHUNKS, _CW)
    dst_p = jnp.concatenate(
        [dst, _N + 600 + (padi % 1024)]).reshape(_NW, _CHUNKS, _CW)

    ones8 = jnp.ones((_CW, 8), f32)
    z8 = jnp.zeros((_TROWS, 8), f32)
    z16 = jnp.zeros((_TROWS, 16), f32)
    z64b = jnp.zeros((_TROWS, 64), bf)

    deg_out = _deg_k(ones8, dst_p, z8)
    dis, xs1 = _dis_kernel(deg_out, xf)

    kt = lambda K, h: K[:, :, h, 0].T.astype(bf)
    rs = lambda v: v.reshape(1, -1)

    agg1 = _agg16(xs1, src_p, dst_p, z16)
    ys2 = _blk1(dis, xs1, agg1, W1.astype(bf), rs(b1), kt(K1, 0), kt(K1, 1),
                kt(K1, 2), rs(kb1))[0]
    ag2 = _agg64b(ys2, src_p, dst_p, z64b)
    ys3a, ys3b = _blk2(dis, ys2, ag2, W2.astype(bf), rs(b2), kt(K2, 0),
                       kt(K2, 1), kt(K2, 2), rs(kb2))
    ag3a = _agg64b(ys3a, src_p, dst_p, z64b)
    ag3b = _agg64b(ys3b, src_p, dst_p, z64b)

    blk3 = _make_block(128, 256, 2, 0, F=Wfc.shape[1])
    out = blk3(dis, ys3a, ys3b, ag3a, ag3b, W3.astype(bf), rs(b3),
               kt(K3, 0), kt(K3, 1), kt(K3, 2), rs(kb3), Wfc, rs(bfc))[0]
    return out
